# baseline (device time: 60333 ns/iter reference)
import jax
import jax.numpy as jnp
from jax import lax
from jax.experimental import pallas as pl
from jax.experimental.pallas import tpu as pltpu

M, N = 2048, 1024
BF16 = jnp.bfloat16

SA, SB = 1792, 256
QA = SA // 4
HA1, HA2 = QA // 2, QA // 4
HB1, HB2 = SB // 2, SB // 4
QB = HB2 // 4


def kernel(x):
    x = x.reshape(M, N)

    def body(x_ref, out_ref, rbxyA, rbxyB, rbzA, rbzB,
             xy_send, xy_recv, z_send, z_recv):
        my_x = lax.axis_index("x")
        my_y = lax.axis_index("y")
        my_z = lax.axis_index("z")
        z_lo = my_z % 2
        z_hi = my_z // 2

        acc = out_ref
        acc[...] = x_ref[...].astype(BF16)

        xy_peers = [
            (1 - my_x, my_y, my_z),
            (my_x, 1 - my_y, my_z),
            (1 - my_x, 1 - my_y, my_z),
        ]
        q_me = my_x * 2 + my_y
        q_peer = [
            (1 - my_x) * 2 + my_y,
            my_x * 2 + (1 - my_y),
            (1 - my_x) * 2 + (1 - my_y),
        ]
        z1_peer = (my_x, my_y, my_z ^ 1)
        z2_peer = (my_x, my_y, my_z ^ 2)

        barrier_sem = pltpu.get_barrier_semaphore()
        for peer in xy_peers + [z1_peer, z2_peer]:
            pl.semaphore_signal(
                barrier_sem, inc=1, device_id=peer,
                device_id_type=pl.DeviceIdType.MESH,
            )
        pl.semaphore_wait(barrier_sem, 5)

        def rcopy(src, dst, ssem, rsem, dev):
            r = pltpu.make_async_remote_copy(
                src_ref=src, dst_ref=dst, send_sem=ssem, recv_sem=rsem,
                device_id=dev, device_id_type=pl.DeviceIdType.MESH,
            )
            r.start()
            return r

        o_fwd1 = (1 - z_lo) * HA1
        o_kept = z_lo * HA1
        o_fwd2 = o_kept + (1 - z_hi) * HA2
        o_fin = o_kept + z_hi * HA2
        offA_q = q_me * QA

        A_SUBS = [(o_fwd1, HA1), (o_fwd2, HA2), (o_fin, HA2)]

        def a_scatter(sub):
            intra, rows = A_SUBS[sub]
            return [
                rcopy(
                    acc.at[pl.ds(q_peer[s] * QA + intra, rows), :],
                    rbxyA.at[pl.ds(s * QA + intra, rows), :],
                    xy_send.at[0, 0, sub, s], xy_recv.at[0, 0, sub, s],
                    xy_peers[s],
                )
                for s in range(3)
            ]

        a_sub0 = a_scatter(0)

        keepB_pk = SA + (1 - z_lo) * HB1
        subB = [(1 - z_hi) * HB2, z_hi * HB2]
        b_z1 = [
            rcopy(
                acc.at[pl.ds(keepB_pk + subB[sub], HB2), :],
                rbzB.at[pl.ds(subB[sub], HB2), :],
                z_send.at[1, 0, 0, sub], z_recv.at[1, 0, 0, sub], z1_peer,
            )
            for sub in (0, 1)
        ]
        offB = SA + z_lo * HB1

        a_sub1 = a_scatter(1)
        a_sub2 = a_scatter(2)

        b_z1[0].wait()
        acc[pl.ds(offB + subB[0], HB2), :] = (
            acc[pl.ds(offB + subB[0], HB2), :] + rbzB[pl.ds(subB[0], HB2), :]
        )
        b_z2 = rcopy(
            acc.at[pl.ds(offB + subB[0], HB2), :],
            rbzB.at[pl.ds(HB1, HB2), :],
            z_send.at[1, 0, 1, 0], z_recv.at[1, 0, 1, 0], z2_peer,
        )

        for r in a_sub0:
            r.wait()
        acc[pl.ds(offA_q + o_fwd1, HA1), :] = (
            acc[pl.ds(offA_q + o_fwd1, HA1), :]
            + rbxyA[pl.ds(0 * QA + o_fwd1, HA1), :]
            + rbxyA[pl.ds(1 * QA + o_fwd1, HA1), :]
            + rbxyA[pl.ds(2 * QA + o_fwd1, HA1), :]
        )
        a_z1 = rcopy(
            acc.at[pl.ds(offA_q + o_fwd1, HA1), :],
            rbzA.at[pl.ds(0, HA1), :],
            z_send.at[0, 0, 0, 0], z_recv.at[0, 0, 0, 0], z1_peer,
        )

        b_z1[1].wait()
        acc[pl.ds(offB + subB[1], HB2), :] = (
            acc[pl.ds(offB + subB[1], HB2), :] + rbzB[pl.ds(subB[1], HB2), :]
        )

        for r in a_sub1:
            r.wait()
        a_z1.wait()
        acc[pl.ds(offA_q + o_fwd2, HA2), :] = (
            acc[pl.ds(offA_q + o_fwd2, HA2), :]
            + rbxyA[pl.ds(0 * QA + o_fwd2, HA2), :]
            + rbxyA[pl.ds(1 * QA + o_fwd2, HA2), :]
            + rbxyA[pl.ds(2 * QA + o_fwd2, HA2), :]
            + rbzA[pl.ds((1 - z_hi) * HA2, HA2), :]
        )
        a_z2 = rcopy(
            acc.at[pl.ds(offA_q + o_fwd2, HA2), :],
            rbzA.at[pl.ds(HA1, HA2), :],
            z_send.at[0, 0, 1, 0], z_recv.at[0, 0, 1, 0], z2_peer,
        )

        b_z2.wait()
        offB = offB + z_hi * HB2
        acc[pl.ds(offB, HB2), :] = (
            acc[pl.ds(offB, HB2), :] + rbzB[pl.ds(HB1, HB2), :]
        )
        b_xy = [
            rcopy(
                acc.at[pl.ds(offB + q_peer[s] * QB, QB), :],
                rbxyB.at[pl.ds(s * QB, QB), :],
                xy_send.at[1, 0, 0, s], xy_recv.at[1, 0, 0, s],
                xy_peers[s],
            )
            for s in range(3)
        ]

        for r in a_sub2:
            r.wait()
        a_z2.wait()
        offFin = offA_q + o_fin
        acc[pl.ds(offFin, HA2), :] = (
            acc[pl.ds(offFin, HA2), :]
            + rbxyA[pl.ds(0 * QA + o_fin, HA2), :]
            + rbxyA[pl.ds(1 * QA + o_fin, HA2), :]
            + rbxyA[pl.ds(2 * QA + o_fin, HA2), :]
            + rbzA[pl.ds(z_hi * HA2, HA2), :]
            + rbzA[pl.ds(HA1, HA2), :]
        )

        def a_bcast(wave, off):
            return [
                rcopy(
                    acc.at[pl.ds(off, HA2), :],
                    acc.at[pl.ds(off, HA2), :],
                    xy_send.at[0, 1, wave, s], xy_recv.at[0, 1, wave, s],
                    xy_peers[s],
                )
                for s in range(3)
            ]

        a_w0 = a_bcast(0, offFin)
        a_agz2 = rcopy(
            acc.at[pl.ds(offFin, HA2), :],
            acc.at[pl.ds(offFin, HA2), :],
            z_send.at[0, 1, 1, 0], z_recv.at[0, 1, 1, 0], z2_peer,
        )
        a_z1a = rcopy(
            acc.at[pl.ds(offFin, HA2), :],
            acc.at[pl.ds(offFin, HA2), :],
            z_send.at[0, 1, 0, 0], z_recv.at[0, 1, 0, 0], z1_peer,
        )

        for r in b_xy:
            r.wait()
        offB2 = offB + q_me * QB
        acc[pl.ds(offB2, QB), :] = (
            acc[pl.ds(offB2, QB), :]
            + rbxyB[pl.ds(0, QB), :]
            + rbxyB[pl.ds(QB, QB), :]
            + rbxyB[pl.ds(2 * QB, QB), :]
        )
        b_xy = [
            rcopy(
                acc.at[pl.ds(offB2, QB), :],
                acc.at[pl.ds(offB2, QB), :],
                xy_send.at[1, 1, 0, s], xy_recv.at[1, 1, 0, s],
                xy_peers[s],
            )
            for s in range(3)
        ]

        a_agz2.wait()
        a_w1 = a_bcast(1, offA_q + o_fwd2)
        a_z1b = rcopy(
            acc.at[pl.ds(offA_q + o_fwd2, HA2), :],
            acc.at[pl.ds(offA_q + o_fwd2, HA2), :],
            z_send.at[0, 1, 0, 1], z_recv.at[0, 1, 0, 1], z1_peer,
        )

        for r in b_xy:
            r.wait()
        b_z2 = rcopy(
            acc.at[pl.ds(offB, HB2), :],
            acc.at[pl.ds(offB, HB2), :],
            z_send.at[1, 1, 1, 0], z_recv.at[1, 1, 1, 0], z2_peer,
        )
        b_z1a = rcopy(
            acc.at[pl.ds(offB, HB2), :],
            acc.at[pl.ds(offB, HB2), :],
            z_send.at[1, 1, 0, 0], z_recv.at[1, 1, 0, 0], z1_peer,
        )

        a_z1a.wait()
        a_w2 = a_bcast(2, offA_q + o_fwd1 + z_hi * HA2)

        b_z2.wait()
        offB_half = SA + z_lo * HB1
        b_z1b = rcopy(
            acc.at[pl.ds(offB_half + (1 - z_hi) * HB2, HB2), :],
            acc.at[pl.ds(offB_half + (1 - z_hi) * HB2, HB2), :],
            z_send.at[1, 1, 0, 1], z_recv.at[1, 1, 0, 1], z1_peer,
        )

        a_z1b.wait()
        a_w3 = a_bcast(3, offA_q + o_fwd1 + (1 - z_hi) * HA2)

        b_z1a.wait()
        b_z1b.wait()
        for wave in (a_w0, a_w1, a_w2, a_w3):
            for r in wave:
                r.wait()

    return pl.pallas_call(
        body,
        out_shape=jax.ShapeDtypeStruct((M, N), BF16),
        in_specs=[pl.BlockSpec(memory_space=pltpu.VMEM)],
        out_specs=pl.BlockSpec(memory_space=pltpu.VMEM),
        scratch_shapes=[
            pltpu.VMEM((3 * QA, N), BF16),
            pltpu.VMEM((3 * QB, N), BF16),
            pltpu.VMEM((HA1 + HA2, N), BF16),
            pltpu.VMEM((HB1 + HB2, N), BF16),
            pltpu.SemaphoreType.DMA((2, 2, 4, 3)),
            pltpu.SemaphoreType.DMA((2, 2, 4, 3)),
            pltpu.SemaphoreType.DMA((2, 2, 2, 2)),
            pltpu.SemaphoreType.DMA((2, 2, 2, 2)),
        ],
        compiler_params=pltpu.CompilerParams(collective_id=0),
    )(x)


# device time: 52446 ns/iter; 1.1504x vs baseline; 1.1504x over previous
import jax
import jax.numpy as jnp
from jax import lax
from jax.experimental import pallas as pl
from jax.experimental.pallas import tpu as pltpu

M, N = 2048, 1024
BF16 = jnp.bfloat16

SA, SB = 1536, 512
QA = SA // 4
HA1, HA2 = QA // 2, QA // 4
HB1, HB2 = SB // 2, SB // 4
QB = HB2 // 4


def kernel(x):
    x = x.reshape(M, N)

    def body(x_ref, out_ref, rbxyA, rbxyB, rbzA, rbzB,
             xy_send, xy_recv, z_send, z_recv):
        my_x = lax.axis_index("x")
        my_y = lax.axis_index("y")
        my_z = lax.axis_index("z")
        z_lo = my_z % 2
        z_hi = my_z // 2

        acc = out_ref
        acc[...] = x_ref[...].astype(BF16)

        xy_peers = [
            (1 - my_x, my_y, my_z),
            (my_x, 1 - my_y, my_z),
            (1 - my_x, 1 - my_y, my_z),
        ]
        q_me = my_x * 2 + my_y
        q_peer = [
            (1 - my_x) * 2 + my_y,
            my_x * 2 + (1 - my_y),
            (1 - my_x) * 2 + (1 - my_y),
        ]
        z1_peer = (my_x, my_y, my_z ^ 1)
        z2_peer = (my_x, my_y, my_z ^ 2)

        barrier_sem = pltpu.get_barrier_semaphore()
        for peer in xy_peers + [z1_peer, z2_peer]:
            pl.semaphore_signal(
                barrier_sem, inc=1, device_id=peer,
                device_id_type=pl.DeviceIdType.MESH,
            )
        pl.semaphore_wait(barrier_sem, 5)

        def rcopy(src, dst, ssem, rsem, dev):
            r = pltpu.make_async_remote_copy(
                src_ref=src, dst_ref=dst, send_sem=ssem, recv_sem=rsem,
                device_id=dev, device_id_type=pl.DeviceIdType.MESH,
            )
            r.start()
            return r

        o_fwd1 = (1 - z_lo) * HA1
        o_kept = z_lo * HA1
        o_fwd2 = o_kept + (1 - z_hi) * HA2
        o_fin = o_kept + z_hi * HA2
        o_f_fwd2 = o_fwd1 + (1 - z_hi) * HA2
        o_f_fin = o_fwd1 + z_hi * HA2
        offA_q = q_me * QA

        A_SUBS = [
            (o_f_fwd2, HA2),
            (o_fwd2, HA2),
            (o_f_fin, HA2),
            (o_fin, HA2),
        ]

        def a_scatter(sub):
            intra, rows = A_SUBS[sub]
            return [
                rcopy(
                    acc.at[pl.ds(q_peer[s] * QA + intra, rows), :],
                    rbxyA.at[pl.ds(s * QA + intra, rows), :],
                    xy_send.at[0, 0, sub, s], xy_recv.at[0, 0, sub, s],
                    xy_peers[s],
                )
                for s in range(3)
            ]

        a_sub0 = a_scatter(0)

        keepB_pk = SA + (1 - z_lo) * HB1
        subB = [(1 - z_hi) * HB2, z_hi * HB2]
        b_z1 = [
            rcopy(
                acc.at[pl.ds(keepB_pk + subB[sub], HB2), :],
                rbzB.at[pl.ds(subB[sub], HB2), :],
                z_send.at[1, 0, 0, sub], z_recv.at[1, 0, 0, sub], z1_peer,
            )
            for sub in (0, 1)
        ]
        offB = SA + z_lo * HB1

        a_sub1 = a_scatter(1)
        a_sub2 = a_scatter(2)
        a_sub3 = a_scatter(3)

        b_z1[0].wait()
        acc[pl.ds(offB + subB[0], HB2), :] = (
            acc[pl.ds(offB + subB[0], HB2), :] + rbzB[pl.ds(subB[0], HB2), :]
        )
        b_z2 = rcopy(
            acc.at[pl.ds(offB + subB[0], HB2), :],
            rbzB.at[pl.ds(HB1, HB2), :],
            z_send.at[1, 0, 1, 0], z_recv.at[1, 0, 1, 0], z2_peer,
        )

        for r in a_sub0:
            r.wait()
        acc[pl.ds(offA_q + o_f_fwd2, HA2), :] = (
            acc[pl.ds(offA_q + o_f_fwd2, HA2), :]
            + rbxyA[pl.ds(0 * QA + o_f_fwd2, HA2), :]
            + rbxyA[pl.ds(1 * QA + o_f_fwd2, HA2), :]
            + rbxyA[pl.ds(2 * QA + o_f_fwd2, HA2), :]
        )
        a_z1a = rcopy(
            acc.at[pl.ds(offA_q + o_f_fwd2, HA2), :],
            rbzA.at[pl.ds((1 - z_hi) * HA2, HA2), :],
            z_send.at[0, 0, 0, 0], z_recv.at[0, 0, 0, 0], z1_peer,
        )

        b_z1[1].wait()
        acc[pl.ds(offB + subB[1], HB2), :] = (
            acc[pl.ds(offB + subB[1], HB2), :] + rbzB[pl.ds(subB[1], HB2), :]
        )

        for r in a_sub1:
            r.wait()
        a_z1a.wait()
        acc[pl.ds(offA_q + o_fwd2, HA2), :] = (
            acc[pl.ds(offA_q + o_fwd2, HA2), :]
            + rbxyA[pl.ds(0 * QA + o_fwd2, HA2), :]
            + rbxyA[pl.ds(1 * QA + o_fwd2, HA2), :]
            + rbxyA[pl.ds(2 * QA + o_fwd2, HA2), :]
            + rbzA[pl.ds((1 - z_hi) * HA2, HA2), :]
        )
        a_z2 = rcopy(
            acc.at[pl.ds(offA_q + o_fwd2, HA2), :],
            rbzA.at[pl.ds(HA1, HA2), :],
            z_send.at[0, 0, 1, 0], z_recv.at[0, 0, 1, 0], z2_peer,
        )

        for r in a_sub2:
            r.wait()
        acc[pl.ds(offA_q + o_f_fin, HA2), :] = (
            acc[pl.ds(offA_q + o_f_fin, HA2), :]
            + rbxyA[pl.ds(0 * QA + o_f_fin, HA2), :]
            + rbxyA[pl.ds(1 * QA + o_f_fin, HA2), :]
            + rbxyA[pl.ds(2 * QA + o_f_fin, HA2), :]
        )
        a_z1b = rcopy(
            acc.at[pl.ds(offA_q + o_f_fin, HA2), :],
            rbzA.at[pl.ds(z_hi * HA2, HA2), :],
            z_send.at[0, 0, 0, 1], z_recv.at[0, 0, 0, 1], z1_peer,
        )

        b_z2.wait()
        offB = offB + z_hi * HB2
        acc[pl.ds(offB, HB2), :] = (
            acc[pl.ds(offB, HB2), :] + rbzB[pl.ds(HB1, HB2), :]
        )
        b_xy = [
            rcopy(
                acc.at[pl.ds(offB + q_peer[s] * QB, QB), :],
                rbxyB.at[pl.ds(s * QB, QB), :],
                xy_send.at[1, 0, 0, s], xy_recv.at[1, 0, 0, s],
                xy_peers[s],
            )
            for s in range(3)
        ]

        for r in a_sub3:
            r.wait()
        a_z1b.wait()
        a_z2.wait()
        offFin = offA_q + o_fin
        acc[pl.ds(offFin, HA2), :] = (
            acc[pl.ds(offFin, HA2), :]
            + rbxyA[pl.ds(0 * QA + o_fin, HA2), :]
            + rbxyA[pl.ds(1 * QA + o_fin, HA2), :]
            + rbxyA[pl.ds(2 * QA + o_fin, HA2), :]
            + rbzA[pl.ds(z_hi * HA2, HA2), :]
            + rbzA[pl.ds(HA1, HA2), :]
        )

        def a_bcast(wave, off):
            return [
                rcopy(
                    acc.at[pl.ds(off, HA2), :],
                    acc.at[pl.ds(off, HA2), :],
                    xy_send.at[0, 1, wave, s], xy_recv.at[0, 1, wave, s],
                    xy_peers[s],
                )
                for s in range(3)
            ]

        a_w0 = a_bcast(0, offFin)
        a_agz2 = rcopy(
            acc.at[pl.ds(offFin, HA2), :],
            acc.at[pl.ds(offFin, HA2), :],
            z_send.at[0, 1, 1, 0], z_recv.at[0, 1, 1, 0], z2_peer,
        )
        a_z1a = rcopy(
            acc.at[pl.ds(offFin, HA2), :],
            acc.at[pl.ds(offFin, HA2), :],
            z_send.at[0, 1, 0, 0], z_recv.at[0, 1, 0, 0], z1_peer,
        )

        for r in b_xy:
            r.wait()
        offB2 = offB + q_me * QB
        acc[pl.ds(offB2, QB), :] = (
            acc[pl.ds(offB2, QB), :]
            + rbxyB[pl.ds(0, QB), :]
            + rbxyB[pl.ds(QB, QB), :]
            + rbxyB[pl.ds(2 * QB, QB), :]
        )
        b_xy = [
            rcopy(
                acc.at[pl.ds(offB2, QB), :],
                acc.at[pl.ds(offB2, QB), :],
                xy_send.at[1, 1, 0, s], xy_recv.at[1, 1, 0, s],
                xy_peers[s],
            )
            for s in range(3)
        ]

        a_agz2.wait()
        a_w1 = a_bcast(1, offA_q + o_fwd2)
        a_z1b = rcopy(
            acc.at[pl.ds(offA_q + o_fwd2, HA2), :],
            acc.at[pl.ds(offA_q + o_fwd2, HA2), :],
            z_send.at[0, 1, 0, 1], z_recv.at[0, 1, 0, 1], z1_peer,
        )

        for r in b_xy:
            r.wait()
        b_z2 = rcopy(
            acc.at[pl.ds(offB, HB2), :],
            acc.at[pl.ds(offB, HB2), :],
            z_send.at[1, 1, 1, 0], z_recv.at[1, 1, 1, 0], z2_peer,
        )
        b_z1a = rcopy(
            acc.at[pl.ds(offB, HB2), :],
            acc.at[pl.ds(offB, HB2), :],
            z_send.at[1, 1, 0, 0], z_recv.at[1, 1, 0, 0], z1_peer,
        )

        a_z1a.wait()
        a_w2 = a_bcast(2, offA_q + o_fwd1 + z_hi * HA2)

        b_z2.wait()
        offB_half = SA + z_lo * HB1
        b_z1b = rcopy(
            acc.at[pl.ds(offB_half + (1 - z_hi) * HB2, HB2), :],
            acc.at[pl.ds(offB_half + (1 - z_hi) * HB2, HB2), :],
            z_send.at[1, 1, 0, 1], z_recv.at[1, 1, 0, 1], z1_peer,
        )

        a_z1b.wait()
        a_w3 = a_bcast(3, offA_q + o_fwd1 + (1 - z_hi) * HA2)

        b_z1a.wait()
        b_z1b.wait()
        for wave in (a_w0, a_w1, a_w2, a_w3):
            for r in wave:
                r.wait()

    return pl.pallas_call(
        body,
        out_shape=jax.ShapeDtypeStruct((M, N), BF16),
        in_specs=[pl.BlockSpec(memory_space=pltpu.VMEM)],
        out_specs=pl.BlockSpec(memory_space=pltpu.VMEM),
        scratch_shapes=[
            pltpu.VMEM((3 * QA, N), BF16),
            pltpu.VMEM((3 * QB, N), BF16),
            pltpu.VMEM((HA1 + HA2, N), BF16),
            pltpu.VMEM((HB1 + HB2, N), BF16),
            pltpu.SemaphoreType.DMA((2, 2, 4, 3)),
            pltpu.SemaphoreType.DMA((2, 2, 4, 3)),
            pltpu.SemaphoreType.DMA((2, 2, 2, 2)),
            pltpu.SemaphoreType.DMA((2, 2, 2, 2)),
        ],
        compiler_params=pltpu.CompilerParams(collective_id=0),
    )(x)
